# Initial kernel scaffold; baseline (speedup 1.0000x reference)
import jax
import jax.numpy as jnp
from jax.experimental import pallas as pl


def kernel(h_paras, E0, nW0, nb0, nW1, nb1, nW2, nb2, cW0, cb0, cW1, cb1, cW2, cb2, kW0, kb0, kW1, kb1, kW2, kb2, eW0, eb0, eW1, eb1, eW2, eb2):
    raise NotImplementedError("write your pallas kernel here")


# speedup vs baseline: 36.7333x; 36.7333x over previous
import functools

import jax
import jax.numpy as jnp
import numpy as np
from jax.experimental import pallas as pl

N = 1000
KNN = 2
RES = 50
MODES = 2
N0 = 1.5
C_EPSILON = 1.0
DX = 0.01
DH = 0.1
H_MIN = 0.2
H_MAX = 0.8
EY_SIZE = 2 * (KNN + 1) * RES
M = MODES * N
NBLK = M // 8
NBLKP = 256
OFFSETS = (-3, -2, -1, 1, 2, 3)

_F32 = jnp.float32
_HIGH = jax.lax.Precision.HIGHEST


def _dot(a, b):
    return jnp.dot(a, b, preferred_element_type=_F32, precision=_HIGH)


def _clampshift(col, o):
    if o == 0:
        return col
    if o > 0:
        return jnp.concatenate(
            [col[o:], jnp.broadcast_to(col[N - 1:N], (o, 1))], axis=0)
    return jnp.concatenate(
        [jnp.broadcast_to(col[0:1], (-o, 1)), col[:N + o]], axis=0)


def _bmm(a, b):
    return jnp.sum(a[:, :, :, None] * b[:, None, :, :], axis=2)


def _bmv(a, v):
    return jnp.sum(a * v[:, None, :], axis=2)


def _eye8():
    r = jax.lax.broadcasted_iota(jnp.int32, (8, 8), 0)
    c = jax.lax.broadcasted_iota(jnp.int32, (8, 8), 1)
    return (r == c).astype(_F32)


def _binv(d):
    bsz = d.shape[0]
    eye8 = _eye8()
    aug = jnp.concatenate([d, jnp.broadcast_to(eye8, (bsz, 8, 8))], axis=2)
    for p in range(8):
        ep = eye8[p]
        piv = aug[:, p, p]
        pivrow = aug[:, p, :] * (1.0 / piv)[:, None]
        fac = aug[:, :, p] - ep[None, :]
        aug = aug - fac[:, :, None] * pivrow[:, None, :]
    return aug[:, :, 8:]


def _shift_down1(x):
    if x.shape[0] == 1:
        return jnp.zeros_like(x)
    return jnp.concatenate([jnp.zeros_like(x[0:1]), x[:-1]], axis=0)


def _shift_up1(x):
    if x.shape[0] == 1:
        return jnp.zeros_like(x)
    return jnp.concatenate([x[1:], jnp.zeros_like(x[0:1])], axis=0)


def _body(h_ref, e0s_ref,
          nW0, nb0, nW1, nb1, nW2, nb2,
          cW0, cb0, cW1, cb1, cW2, cb2,
          kW0, kb0, kW1, kb1, kW2, kb2,
          eW0, eb0, eW1, eb1, eW2, eb2,
          out_ref):
    hs = jax.nn.sigmoid(h_ref[:]) * (H_MAX - H_MIN) + H_MIN

    h = jnp.tanh(hs * nW0[:] + nb0[:])
    h = jnp.tanh(_dot(h, nW1[:]) + nb1[:])
    neff = _dot(h, nW2[:]) + nb2[:]

    h = jnp.tanh(hs * eW0[:] + eb0[:])
    h = jnp.tanh(_dot(h, eW1[:]) + eb1[:])
    ey = _dot(h, eW2[:]) + eb2[:]
    ey0 = ey[:, :EY_SIZE]
    ey1 = ey[:, EY_SIZE:]
    e0s = e0s_ref[:]
    es0 = jnp.sum(ey0 * e0s, axis=1, keepdims=True)
    es1 = jnp.sum(ey1 * e0s, axis=1, keepdims=True)
    e_sum = jnp.concatenate([es0, es1], axis=1)
    eta = neff * N0 / (neff + N0)
    u0 = (2.0 * C_EPSILON * DX) * eta * e_sum

    hsh = [_clampshift(hs, o) for o in OFFSETS]

    def edge_mlp(W0, b0, W1, b1, W2, b2):
        xs = []
        for oi, o in enumerate(OFFSETS):
            pre = hs * W0[0:1, :] + hsh[oi] * W0[1:2, :] \
                + (float(o) * W0[2:3, :] + b0[:])
            xs.append(jnp.tanh(pre))
        x = jnp.concatenate(xs, axis=0)
        x = jnp.tanh(_dot(x, W1[:]) + b1[:])
        v = _dot(x, W2[:]) + b2[:]
        return [v[oi * N:(oi + 1) * N] for oi in range(len(OFFSETS))]

    cv = edge_mlp(cW0, cb0, cW1, cb1, cW2, cb2)
    kv = edge_mlp(kW0, kb0, kW1, kb1, kW2, kb2)

    ycols = [jnp.zeros((N, 1), _F32), jnp.zeros((N, 1), _F32)]
    ucols = [u0[:, 0:1], u0[:, 1:2]]
    for oi, o in enumerate(OFFSETS):
        for mi in range(2):
            for mj in range(2):
                ycols[mi] = ycols[mi] + kv[oi][:, 2 * mi + mj:2 * mi + mj + 1] \
                    * _clampshift(ucols[mj], o)
    yk = jnp.concatenate(ycols, axis=1)

    rq = jax.lax.broadcasted_iota(jnp.int32, (NBLK, N), 0)
    cn = jax.lax.broadcasted_iota(jnp.int32, (NBLK, N), 1)
    sel = [(cn == 4 * rq + s).astype(_F32) for s in range(4)]
    b = jnp.concatenate([_dot(sel[s], yk) for s in range(4)], axis=1)

    zcol = jnp.zeros((N, 1), _F32)
    cols = {}
    for oi, o in enumerate(OFFSETS):
        for mi in range(2):
            for mj in range(2):
                key = (mi, 2 * o + mj - mi + 7)
                v = cv[oi][:, 2 * mi + mj:2 * mi + mj + 1]
                cols[key] = cols[key] + v if key in cols else v
    band_mi = []
    for mi in range(2):
        band_mi.append(jnp.concatenate(
            [cols.get((mi, di), zcol) for di in range(15)], axis=1))

    drows, lrows, urows = [], [], []
    for a in range(8):
        ba = _dot(sel[a // 2], band_mi[a % 2])
        drows.append(ba[:, 7 - a:15 - a])
        lw = 7 - a
        if lw <= 0:
            lrows.append(jnp.zeros((NBLK, 8), _F32))
        else:
            lrows.append(jnp.concatenate(
                [jnp.zeros((NBLK, a + 1), _F32), ba[:, 0:lw]], axis=1))
        if a == 0:
            urows.append(jnp.zeros((NBLK, 8), _F32))
        else:
            urows.append(jnp.concatenate(
                [ba[:, 15 - a:15], jnp.zeros((NBLK, 8 - a), _F32)], axis=1))
    eye8 = _eye8()
    dm = jnp.stack(drows, axis=1) + eye8[None]
    lm = jnp.stack(lrows, axis=1)
    um = jnp.stack(urows, axis=1)

    def _r22(x):
        return jnp.concatenate([x[:, 0:2], x[:, 2:4]], axis=0)

    pb0 = _r22(cv[0][0:1] + cv[1][0:1] + cv[2][0:1])
    pb1 = _r22(cv[0][1:2] + cv[1][1:2])
    pb2 = _r22(cv[0][2:3])
    pbot = jnp.concatenate([pb0, pb1, pb2], axis=0)
    patch0 = jnp.concatenate(
        [jnp.concatenate([pbot, jnp.zeros((6, 6), _F32)], axis=1),
         jnp.zeros((2, 8), _F32)], axis=0)
    pt0 = _r22(cv[5][N - 3:N - 2])
    pt1 = _r22(cv[4][N - 2:N - 1] + cv[5][N - 2:N - 1])
    pt2 = _r22(cv[3][N - 1:N] + cv[4][N - 1:N] + cv[5][N - 1:N])
    ptop = jnp.concatenate([pt0, pt1, pt2], axis=0)
    patch9 = jnp.concatenate(
        [jnp.zeros((2, 8), _F32),
         jnp.concatenate([jnp.zeros((6, 6), _F32), ptop], axis=1)], axis=0)
    dm = jnp.concatenate(
        [dm[0:1] + patch0[None], dm[1:NBLK - 1], dm[NBLK - 1:] + patch9[None]],
        axis=0)

    pad = NBLKP - NBLK
    dm = jnp.concatenate([dm, jnp.broadcast_to(eye8, (pad, 8, 8))], axis=0)
    lm = jnp.concatenate([lm, jnp.zeros((pad, 8, 8), _F32)], axis=0)
    um = jnp.concatenate([um, jnp.zeros((pad, 8, 8), _F32)], axis=0)
    b = jnp.concatenate([b, jnp.zeros((pad, 8), _F32)], axis=0)
    lm = jnp.concatenate([jnp.zeros((1, 8, 8), _F32), lm[1:]], axis=0)
    um = jnp.concatenate(
        [um[:NBLK - 1], jnp.zeros((1, 8, 8), _F32), um[NBLK:]], axis=0)

    levels = []
    while dm.shape[0] > 1:
        half = dm.shape[0] // 2
        dr = dm.reshape(half, 2, 8, 8)
        lr = lm.reshape(half, 2, 8, 8)
        ur = um.reshape(half, 2, 8, 8)
        br = b.reshape(half, 2, 8)
        ed, od = dr[:, 0], dr[:, 1]
        el, ol = lr[:, 0], lr[:, 1]
        eu, ou = ur[:, 0], ur[:, 1]
        eb, ob = br[:, 0], br[:, 1]
        inv_o = _binv(od)
        p = _bmm(el, _shift_down1(inv_o))
        q = _bmm(eu, inv_o)
        dm = ed - _bmm(p, _shift_down1(ou)) - _bmm(q, ol)
        lm = -_bmm(p, _shift_down1(ol))
        um = -_bmm(q, ou)
        b = eb - _bmv(p, _shift_down1(ob)) - _bmv(q, ob)
        levels.append((ol, ou, ob, inv_o))
    x = _bmv(_binv(dm), b)
    for (ol, ou, ob, inv_o) in reversed(levels):
        rhs = ob - _bmv(ol, x) - _bmv(ou, _shift_up1(x))
        xo = _bmv(inv_o, rhs)
        x = jnp.stack([x, xo], axis=1).reshape(-1, 8)
    xr = x[:NBLK]
    rn = jax.lax.broadcasted_iota(jnp.int32, (N, NBLK), 0)
    cq = jax.lax.broadcasted_iota(jnp.int32, (N, NBLK), 1)
    selt = [(rn == 4 * cq + s).astype(_F32) for s in range(4)]
    du_cols = []
    for mi in range(2):
        dmi = jnp.zeros((N, 1), _F32)
        for s in range(4):
            dmi = dmi + _dot(selt[s], xr[:, 2 * s + mi:2 * s + mi + 1])
        du_cols.append(dmi)
    du = jnp.concatenate(du_cols, axis=1)

    u = u0 + DH * du

    w = eta * u
    contrib = w[:, 0:1] * ey0 + w[:, 1:2] * ey1
    acc = jnp.zeros((N + 5, RES), _F32)
    for t in range(6):
        piece = contrib[:, t * RES:(t + 1) * RES]
        parts = []
        if t > 0:
            parts.append(jnp.zeros((t, RES), _F32))
        parts.append(piece)
        if t < 5:
            parts.append(jnp.zeros((5 - t, RES), _F32))
        acc = acc + jnp.concatenate(parts, axis=0)
    out_ref[:] = acc


@jax.jit
def kernel(h_paras, E0, nW0, nb0, nW1, nb1, nW2, nb2,
           cW0, cb0, cW1, cb1, cW2, cb2,
           kW0, kb0, kW1, kb1, kW2, kb2,
           eW0, eb0, eW1, eb1, eW2, eb2):
    pad1 = (2 * KNN + 1) * RES // 2
    e0p = jnp.pad(E0, (pad1, (2 * KNN + 1) * RES - pad1))
    e0s = jnp.concatenate(
        [e0p[i * RES:(N + i) * RES].reshape(N, RES)
         for i in range(2 * (KNN + 1))], axis=1)

    args = [h_paras.reshape(N, 1), e0s,
            nW0, nb0.reshape(1, -1), nW1, nb1.reshape(1, -1),
            nW2, nb2.reshape(1, -1),
            cW0, cb0.reshape(1, -1), cW1, cb1.reshape(1, -1),
            cW2, cb2.reshape(1, -1),
            kW0, kb0.reshape(1, -1), kW1, kb1.reshape(1, -1),
            kW2, kb2.reshape(1, -1),
            eW0, eb0.reshape(1, -1), eW1, eb1.reshape(1, -1),
            eW2, eb2.reshape(1, -1)]

    acc = pl.pallas_call(
        _body,
        out_shape=jax.ShapeDtypeStruct((N + 5, RES), jnp.float32),
    )(*args)

    start = (2 * KNN + 1) * RES // 2
    return acc.reshape(-1)[start:start + N * RES]


# speedup vs baseline: 43.8120x; 1.1927x over previous
import functools

import jax
import jax.numpy as jnp
import numpy as np
from jax.experimental import pallas as pl

N = 1000
KNN = 2
RES = 50
MODES = 2
N0 = 1.5
C_EPSILON = 1.0
DX = 0.01
DH = 0.1
H_MIN = 0.2
H_MAX = 0.8
EY_SIZE = 2 * (KNN + 1) * RES
M = MODES * N
NBLK = M // 8
NBLKP = 256
OFFSETS = (-3, -2, -1, 1, 2, 3)

_F32 = jnp.float32
_HIGH = jax.lax.Precision.HIGHEST


def _dot(a, b):
    return jnp.dot(a, b, preferred_element_type=_F32, precision=_HIGH)


def _clampshift(col, o):
    if o == 0:
        return col
    if o > 0:
        return jnp.concatenate(
            [col[o:], jnp.broadcast_to(col[N - 1:N], (o, 1))], axis=0)
    return jnp.concatenate(
        [jnp.broadcast_to(col[0:1], (-o, 1)), col[:N + o]], axis=0)


def _bmm(a, b):
    out = a[:, :, 0:1] * b[:, 0:1, :]
    for k in range(1, 8):
        out = out + a[:, :, k:k + 1] * b[:, k:k + 1, :]
    return out


def _bmv(a, v):
    return jnp.sum(a * v[:, None, :], axis=2)


def _eye8():
    r = jax.lax.broadcasted_iota(jnp.int32, (8, 8), 0)
    c = jax.lax.broadcasted_iota(jnp.int32, (8, 8), 1)
    return (r == c).astype(_F32)


def _binv(d):
    bsz = d.shape[0]
    eye8 = _eye8()
    aug = jnp.concatenate([d, jnp.broadcast_to(eye8, (bsz, 8, 8))], axis=2)
    for p in range(8):
        ep = eye8[p]
        piv = aug[:, p, p]
        pivrow = aug[:, p, :] * (1.0 / piv)[:, None]
        fac = aug[:, :, p] - ep[None, :]
        aug = aug - fac[:, :, None] * pivrow[:, None, :]
    return aug[:, :, 8:]


def _shift_down1(x):
    if x.shape[0] == 1:
        return jnp.zeros_like(x)
    return jnp.concatenate([jnp.zeros_like(x[0:1]), x[:-1]], axis=0)


def _shift_up1(x):
    if x.shape[0] == 1:
        return jnp.zeros_like(x)
    return jnp.concatenate([x[1:], jnp.zeros_like(x[0:1])], axis=0)


def _body(h_ref, e0s_ref,
          nW0, nb0, nW1, nb1, nW2, nb2,
          cW0, cb0, cW1, cb1, cW2, cb2,
          kW0, kb0, kW1, kb1, kW2, kb2,
          eW0, eb0, eW1, eb1, eW2, eb2,
          out_ref):
    hs = jax.nn.sigmoid(h_ref[:]) * (H_MAX - H_MIN) + H_MIN

    h = jnp.tanh(hs * nW0[:] + nb0[:])
    h = jnp.tanh(_dot(h, nW1[:]) + nb1[:])
    neff = _dot(h, nW2[:]) + nb2[:]

    h = jnp.tanh(hs * eW0[:] + eb0[:])
    h = jnp.tanh(_dot(h, eW1[:]) + eb1[:])
    ey = _dot(h, eW2[:]) + eb2[:]
    ey0 = ey[:, :EY_SIZE]
    ey1 = ey[:, EY_SIZE:]
    e0s = e0s_ref[:]
    es0 = jnp.sum(ey0 * e0s, axis=1, keepdims=True)
    es1 = jnp.sum(ey1 * e0s, axis=1, keepdims=True)
    e_sum = jnp.concatenate([es0, es1], axis=1)
    eta = neff * N0 / (neff + N0)
    u0 = (2.0 * C_EPSILON * DX) * eta * e_sum

    hsh = [_clampshift(hs, o) for o in OFFSETS]

    def edge_mlp(W0, b0, W1, b1, W2, b2):
        xs = []
        for oi, o in enumerate(OFFSETS):
            pre = hs * W0[0:1, :] + hsh[oi] * W0[1:2, :] \
                + (float(o) * W0[2:3, :] + b0[:])
            xs.append(jnp.tanh(pre))
        x = jnp.concatenate(xs, axis=0)
        x = jnp.tanh(_dot(x, W1[:]) + b1[:])
        v = _dot(x, W2[:]) + b2[:]
        return [v[oi * N:(oi + 1) * N] for oi in range(len(OFFSETS))]

    cv = edge_mlp(cW0, cb0, cW1, cb1, cW2, cb2)
    kv = edge_mlp(kW0, kb0, kW1, kb1, kW2, kb2)

    ycols = [jnp.zeros((N, 1), _F32), jnp.zeros((N, 1), _F32)]
    ucols = [u0[:, 0:1], u0[:, 1:2]]
    for oi, o in enumerate(OFFSETS):
        for mi in range(2):
            for mj in range(2):
                ycols[mi] = ycols[mi] + kv[oi][:, 2 * mi + mj:2 * mi + mj + 1] \
                    * _clampshift(ucols[mj], o)
    yk = jnp.concatenate(ycols, axis=1)

    rmod4 = jax.lax.broadcasted_iota(jnp.int32, (N, 1), 0) % 4

    def _pick4(x, s):
        w = x.shape[1]
        masked = x * (rmod4 == s).astype(_F32)
        return jnp.sum(masked.reshape(NBLK, 4, w), axis=1)

    b = jnp.concatenate([_pick4(yk, s) for s in range(4)], axis=1)

    zcol = jnp.zeros((N, 1), _F32)
    cols = {}
    for oi, o in enumerate(OFFSETS):
        for mi in range(2):
            for mj in range(2):
                key = (mi, 2 * o + mj - mi + 7)
                v = cv[oi][:, 2 * mi + mj:2 * mi + mj + 1]
                cols[key] = cols[key] + v if key in cols else v
    band_mi = []
    for mi in range(2):
        band_mi.append(jnp.concatenate(
            [cols.get((mi, di), zcol) for di in range(15)], axis=1))

    drows, lrows, urows = [], [], []
    for a in range(8):
        ba = _pick4(band_mi[a % 2], a // 2)
        drows.append(ba[:, 7 - a:15 - a])
        lw = 7 - a
        if lw <= 0:
            lrows.append(jnp.zeros((NBLK, 8), _F32))
        else:
            lrows.append(jnp.concatenate(
                [jnp.zeros((NBLK, a + 1), _F32), ba[:, 0:lw]], axis=1))
        if a == 0:
            urows.append(jnp.zeros((NBLK, 8), _F32))
        else:
            urows.append(jnp.concatenate(
                [ba[:, 15 - a:15], jnp.zeros((NBLK, 8 - a), _F32)], axis=1))
    eye8 = _eye8()
    dm = jnp.stack(drows, axis=1) + eye8[None]
    lm = jnp.stack(lrows, axis=1)
    um = jnp.stack(urows, axis=1)

    def _r22(x):
        return jnp.concatenate([x[:, 0:2], x[:, 2:4]], axis=0)

    pb0 = _r22(cv[0][0:1] + cv[1][0:1] + cv[2][0:1])
    pb1 = _r22(cv[0][1:2] + cv[1][1:2])
    pb2 = _r22(cv[0][2:3])
    pbot = jnp.concatenate([pb0, pb1, pb2], axis=0)
    patch0 = jnp.concatenate(
        [jnp.concatenate([pbot, jnp.zeros((6, 6), _F32)], axis=1),
         jnp.zeros((2, 8), _F32)], axis=0)
    pt0 = _r22(cv[5][N - 3:N - 2])
    pt1 = _r22(cv[4][N - 2:N - 1] + cv[5][N - 2:N - 1])
    pt2 = _r22(cv[3][N - 1:N] + cv[4][N - 1:N] + cv[5][N - 1:N])
    ptop = jnp.concatenate([pt0, pt1, pt2], axis=0)
    patch9 = jnp.concatenate(
        [jnp.zeros((2, 8), _F32),
         jnp.concatenate([jnp.zeros((6, 6), _F32), ptop], axis=1)], axis=0)
    dm = jnp.concatenate(
        [dm[0:1] + patch0[None], dm[1:NBLK - 1], dm[NBLK - 1:] + patch9[None]],
        axis=0)

    pad = NBLKP - NBLK
    dm = jnp.concatenate([dm, jnp.broadcast_to(eye8, (pad, 8, 8))], axis=0)
    lm = jnp.concatenate([lm, jnp.zeros((pad, 8, 8), _F32)], axis=0)
    um = jnp.concatenate([um, jnp.zeros((pad, 8, 8), _F32)], axis=0)
    b = jnp.concatenate([b, jnp.zeros((pad, 8), _F32)], axis=0)
    lm = jnp.concatenate([jnp.zeros((1, 8, 8), _F32), lm[1:]], axis=0)
    um = jnp.concatenate(
        [um[:NBLK - 1], jnp.zeros((1, 8, 8), _F32), um[NBLK:]], axis=0)

    levels = []
    while dm.shape[0] > 1:
        half = dm.shape[0] // 2
        dr = dm.reshape(half, 2, 8, 8)
        lr = lm.reshape(half, 2, 8, 8)
        ur = um.reshape(half, 2, 8, 8)
        br = b.reshape(half, 2, 8)
        ed, od = dr[:, 0], dr[:, 1]
        el, ol = lr[:, 0], lr[:, 1]
        eu, ou = ur[:, 0], ur[:, 1]
        eb, ob = br[:, 0], br[:, 1]
        inv_o = _binv(od)
        p = _bmm(el, _shift_down1(inv_o))
        q = _bmm(eu, inv_o)
        dm = ed - _bmm(p, _shift_down1(ou)) - _bmm(q, ol)
        lm = -_bmm(p, _shift_down1(ol))
        um = -_bmm(q, ou)
        b = eb - _bmv(p, _shift_down1(ob)) - _bmv(q, ob)
        levels.append((ol, ou, ob, inv_o))
    x = _bmv(_binv(dm), b)
    for (ol, ou, ob, inv_o) in reversed(levels):
        rhs = ob - _bmv(ol, x) - _bmv(ou, _shift_up1(x))
        xo = _bmv(inv_o, rhs)
        x = jnp.stack([x, xo], axis=1).reshape(-1, 8)
    xr = x[:NBLK]
    rep8 = jnp.repeat(xr, 4, axis=0)
    rmod = jax.lax.broadcasted_iota(jnp.int32, (N, 8), 0) % 4
    lane = jax.lax.broadcasted_iota(jnp.int32, (N, 8), 1)
    du_cols = []
    for mi in range(2):
        mask = (lane == 2 * rmod + mi).astype(_F32)
        du_cols.append(jnp.sum(rep8 * mask, axis=1, keepdims=True))
    du = jnp.concatenate(du_cols, axis=1)

    u = u0 + DH * du

    w = eta * u
    contrib = w[:, 0:1] * ey0 + w[:, 1:2] * ey1
    acc = jnp.zeros((N + 5, RES), _F32)
    for t in range(6):
        piece = contrib[:, t * RES:(t + 1) * RES]
        parts = []
        if t > 0:
            parts.append(jnp.zeros((t, RES), _F32))
        parts.append(piece)
        if t < 5:
            parts.append(jnp.zeros((5 - t, RES), _F32))
        acc = acc + jnp.concatenate(parts, axis=0)
    out_ref[:] = acc


@jax.jit
def kernel(h_paras, E0, nW0, nb0, nW1, nb1, nW2, nb2,
           cW0, cb0, cW1, cb1, cW2, cb2,
           kW0, kb0, kW1, kb1, kW2, kb2,
           eW0, eb0, eW1, eb1, eW2, eb2):
    pad1 = (2 * KNN + 1) * RES // 2
    e0p = jnp.pad(E0, (pad1, (2 * KNN + 1) * RES - pad1))
    e0s = jnp.concatenate(
        [e0p[i * RES:(N + i) * RES].reshape(N, RES)
         for i in range(2 * (KNN + 1))], axis=1)

    args = [h_paras.reshape(N, 1), e0s,
            nW0, nb0.reshape(1, -1), nW1, nb1.reshape(1, -1),
            nW2, nb2.reshape(1, -1),
            cW0, cb0.reshape(1, -1), cW1, cb1.reshape(1, -1),
            cW2, cb2.reshape(1, -1),
            kW0, kb0.reshape(1, -1), kW1, kb1.reshape(1, -1),
            kW2, kb2.reshape(1, -1),
            eW0, eb0.reshape(1, -1), eW1, eb1.reshape(1, -1),
            eW2, eb2.reshape(1, -1)]

    acc = pl.pallas_call(
        _body,
        out_shape=jax.ShapeDtypeStruct((N + 5, RES), jnp.float32),
    )(*args)

    start = (2 * KNN + 1) * RES // 2
    return acc.reshape(-1)[start:start + N * RES]


# speedup vs baseline: 50.1993x; 1.1458x over previous
import functools

import jax
import jax.numpy as jnp
import numpy as np
from jax.experimental import pallas as pl

N = 1000
KNN = 2
RES = 50
MODES = 2
N0 = 1.5
C_EPSILON = 1.0
DX = 0.01
DH = 0.1
H_MIN = 0.2
H_MAX = 0.8
EY_SIZE = 2 * (KNN + 1) * RES
M = MODES * N
NBLK = M // 8
NBLKP = 256
OFFSETS = (-3, -2, -1, 1, 2, 3)

_F32 = jnp.float32
_HIGH = jax.lax.Precision.HIGHEST


def _dot(a, b):
    return jnp.dot(a, b, preferred_element_type=_F32, precision=_HIGH)


def _dotb(a, b):
    return jnp.dot(a.astype(jnp.bfloat16), b.astype(jnp.bfloat16),
                   preferred_element_type=_F32)


def _clampshift(col, o):
    if o == 0:
        return col
    if o > 0:
        return jnp.concatenate(
            [col[o:], jnp.broadcast_to(col[N - 1:N], (o, 1))], axis=0)
    return jnp.concatenate(
        [jnp.broadcast_to(col[0:1], (-o, 1)), col[:N + o]], axis=0)


def _bmm(a, b):
    out = a[:, :, 0:1] * b[:, 0:1, :]
    for k in range(1, 8):
        out = out + a[:, :, k:k + 1] * b[:, k:k + 1, :]
    return out


def _bmv(a, v):
    return jnp.sum(a * v[:, None, :], axis=2)


def _eye8():
    r = jax.lax.broadcasted_iota(jnp.int32, (8, 8), 0)
    c = jax.lax.broadcasted_iota(jnp.int32, (8, 8), 1)
    return (r == c).astype(_F32)


def _binv(d):
    bsz = d.shape[0]
    eye8 = _eye8()
    aug = jnp.concatenate([d, jnp.broadcast_to(eye8, (bsz, 8, 8))], axis=2)
    for p in range(8):
        ep = eye8[p]
        piv = aug[:, p, p]
        pivrow = aug[:, p, :] * (1.0 / piv)[:, None]
        fac = aug[:, :, p] - ep[None, :]
        aug = aug - fac[:, :, None] * pivrow[:, None, :]
    return aug[:, :, 8:]


def _shift_down1(x):
    if x.shape[0] == 1:
        return jnp.zeros_like(x)
    return jnp.concatenate([jnp.zeros_like(x[0:1]), x[:-1]], axis=0)


def _shift_up1(x):
    if x.shape[0] == 1:
        return jnp.zeros_like(x)
    return jnp.concatenate([x[1:], jnp.zeros_like(x[0:1])], axis=0)


def _body(h_ref, e0s_ref,
          nW0, nb0, nW1, nb1, nW2, nb2,
          cW0, cb0, cW1, cb1, cW2, cb2,
          kW0, kb0, kW1, kb1, kW2, kb2,
          eW0, eb0, eW1, eb1, eW2, eb2,
          out_ref):
    hs = jax.nn.sigmoid(h_ref[:]) * (H_MAX - H_MIN) + H_MIN

    h = jnp.tanh(hs * nW0[:] + nb0[:])
    h = jnp.tanh(_dotb(h, nW1[:]) + nb1[:])
    neff = _dotb(h, nW2[:]) + nb2[:]

    h = jnp.tanh(hs * eW0[:] + eb0[:])
    h = jnp.tanh(_dotb(h, eW1[:]) + eb1[:])
    ey = _dotb(h, eW2[:]) + eb2[:]
    ey0 = ey[:, :EY_SIZE]
    ey1 = ey[:, EY_SIZE:]
    e0s = e0s_ref[:]
    es0 = jnp.sum(ey0 * e0s, axis=1, keepdims=True)
    es1 = jnp.sum(ey1 * e0s, axis=1, keepdims=True)
    e_sum = jnp.concatenate([es0, es1], axis=1)
    eta = neff * N0 / (neff + N0)
    u0 = (2.0 * C_EPSILON * DX) * eta * e_sum

    hsh = [_clampshift(hs, o) for o in OFFSETS]

    def edge_mlp(W0, b0, W1, b1, W2, b2):
        xs = []
        for oi, o in enumerate(OFFSETS):
            pre = hs * W0[0:1, :] + hsh[oi] * W0[1:2, :] \
                + (float(o) * W0[2:3, :] + b0[:])
            xs.append(jnp.tanh(pre))
        x = jnp.concatenate(xs, axis=0)
        x = jnp.tanh(_dotb(x, W1[:]) + b1[:])
        v = _dotb(x, W2[:]) + b2[:]
        return [v[oi * N:(oi + 1) * N] for oi in range(len(OFFSETS))]

    cv = edge_mlp(cW0, cb0, cW1, cb1, cW2, cb2)
    kv = edge_mlp(kW0, kb0, kW1, kb1, kW2, kb2)

    ycols = [jnp.zeros((N, 1), _F32), jnp.zeros((N, 1), _F32)]
    ucols = [u0[:, 0:1], u0[:, 1:2]]
    for oi, o in enumerate(OFFSETS):
        for mi in range(2):
            for mj in range(2):
                ycols[mi] = ycols[mi] + kv[oi][:, 2 * mi + mj:2 * mi + mj + 1] \
                    * _clampshift(ucols[mj], o)
    yk = jnp.concatenate(ycols, axis=1)

    rmod4 = jax.lax.broadcasted_iota(jnp.int32, (N, 1), 0) % 4

    def _pick4(x, s):
        w = x.shape[1]
        masked = x * (rmod4 == s).astype(_F32)
        return jnp.sum(masked.reshape(NBLK, 4, w), axis=1)

    b = jnp.concatenate([_pick4(yk, s) for s in range(4)], axis=1)

    zcol = jnp.zeros((N, 1), _F32)
    cols = {}
    for oi, o in enumerate(OFFSETS):
        for mi in range(2):
            for mj in range(2):
                key = (mi, 2 * o + mj - mi + 7)
                v = cv[oi][:, 2 * mi + mj:2 * mi + mj + 1]
                cols[key] = cols[key] + v if key in cols else v
    band_mi = []
    for mi in range(2):
        band_mi.append(jnp.concatenate(
            [cols.get((mi, di), zcol) for di in range(15)], axis=1))

    drows, lrows, urows = [], [], []
    for a in range(8):
        ba = _pick4(band_mi[a % 2], a // 2)
        drows.append(ba[:, 7 - a:15 - a])
        lw = 7 - a
        if lw <= 0:
            lrows.append(jnp.zeros((NBLK, 8), _F32))
        else:
            lrows.append(jnp.concatenate(
                [jnp.zeros((NBLK, a + 1), _F32), ba[:, 0:lw]], axis=1))
        if a == 0:
            urows.append(jnp.zeros((NBLK, 8), _F32))
        else:
            urows.append(jnp.concatenate(
                [ba[:, 15 - a:15], jnp.zeros((NBLK, 8 - a), _F32)], axis=1))
    eye8 = _eye8()
    dm = jnp.stack(drows, axis=1) + eye8[None]
    lm = jnp.stack(lrows, axis=1)
    um = jnp.stack(urows, axis=1)

    def _r22(x):
        return jnp.concatenate([x[:, 0:2], x[:, 2:4]], axis=0)

    pb0 = _r22(cv[0][0:1] + cv[1][0:1] + cv[2][0:1])
    pb1 = _r22(cv[0][1:2] + cv[1][1:2])
    pb2 = _r22(cv[0][2:3])
    pbot = jnp.concatenate([pb0, pb1, pb2], axis=0)
    patch0 = jnp.concatenate(
        [jnp.concatenate([pbot, jnp.zeros((6, 6), _F32)], axis=1),
         jnp.zeros((2, 8), _F32)], axis=0)
    pt0 = _r22(cv[5][N - 3:N - 2])
    pt1 = _r22(cv[4][N - 2:N - 1] + cv[5][N - 2:N - 1])
    pt2 = _r22(cv[3][N - 1:N] + cv[4][N - 1:N] + cv[5][N - 1:N])
    ptop = jnp.concatenate([pt0, pt1, pt2], axis=0)
    patch9 = jnp.concatenate(
        [jnp.zeros((2, 8), _F32),
         jnp.concatenate([jnp.zeros((6, 6), _F32), ptop], axis=1)], axis=0)
    dm = jnp.concatenate(
        [dm[0:1] + patch0[None], dm[1:NBLK - 1], dm[NBLK - 1:] + patch9[None]],
        axis=0)

    pad = NBLKP - NBLK
    dm = jnp.concatenate([dm, jnp.broadcast_to(eye8, (pad, 8, 8))], axis=0)
    lm = jnp.concatenate([lm, jnp.zeros((pad, 8, 8), _F32)], axis=0)
    um = jnp.concatenate([um, jnp.zeros((pad, 8, 8), _F32)], axis=0)
    b = jnp.concatenate([b, jnp.zeros((pad, 8), _F32)], axis=0)
    lm = jnp.concatenate([jnp.zeros((1, 8, 8), _F32), lm[1:]], axis=0)
    um = jnp.concatenate(
        [um[:NBLK - 1], jnp.zeros((1, 8, 8), _F32), um[NBLK:]], axis=0)

    levels = []
    while dm.shape[0] > 1:
        half = dm.shape[0] // 2
        dr = dm.reshape(half, 2, 8, 8)
        lr = lm.reshape(half, 2, 8, 8)
        ur = um.reshape(half, 2, 8, 8)
        br = b.reshape(half, 2, 8)
        ed, od = dr[:, 0], dr[:, 1]
        el, ol = lr[:, 0], lr[:, 1]
        eu, ou = ur[:, 0], ur[:, 1]
        eb, ob = br[:, 0], br[:, 1]
        inv_o = _binv(od)
        p = _bmm(el, _shift_down1(inv_o))
        q = _bmm(eu, inv_o)
        dm = ed - _bmm(p, _shift_down1(ou)) - _bmm(q, ol)
        lm = -_bmm(p, _shift_down1(ol))
        um = -_bmm(q, ou)
        b = eb - _bmv(p, _shift_down1(ob)) - _bmv(q, ob)
        levels.append((ol, ou, ob, inv_o))
    x = _bmv(_binv(dm), b)
    for (ol, ou, ob, inv_o) in reversed(levels):
        rhs = ob - _bmv(ol, x) - _bmv(ou, _shift_up1(x))
        xo = _bmv(inv_o, rhs)
        x = jnp.stack([x, xo], axis=1).reshape(-1, 8)
    xr = x[:NBLK]
    rep8 = jnp.repeat(xr, 4, axis=0)
    rmod = jax.lax.broadcasted_iota(jnp.int32, (N, 8), 0) % 4
    lane = jax.lax.broadcasted_iota(jnp.int32, (N, 8), 1)
    du_cols = []
    for mi in range(2):
        mask = (lane == 2 * rmod + mi).astype(_F32)
        du_cols.append(jnp.sum(rep8 * mask, axis=1, keepdims=True))
    du = jnp.concatenate(du_cols, axis=1)

    u = u0 + DH * du

    w = eta * u
    contrib = w[:, 0:1] * ey0 + w[:, 1:2] * ey1
    acc = jnp.zeros((N + 5, RES), _F32)
    for t in range(6):
        piece = contrib[:, t * RES:(t + 1) * RES]
        parts = []
        if t > 0:
            parts.append(jnp.zeros((t, RES), _F32))
        parts.append(piece)
        if t < 5:
            parts.append(jnp.zeros((5 - t, RES), _F32))
        acc = acc + jnp.concatenate(parts, axis=0)
    out_ref[:] = acc


@jax.jit
def kernel(h_paras, E0, nW0, nb0, nW1, nb1, nW2, nb2,
           cW0, cb0, cW1, cb1, cW2, cb2,
           kW0, kb0, kW1, kb1, kW2, kb2,
           eW0, eb0, eW1, eb1, eW2, eb2):
    pad1 = (2 * KNN + 1) * RES // 2
    e0p = jnp.pad(E0, (pad1, (2 * KNN + 1) * RES - pad1))
    e0s = jnp.concatenate(
        [e0p[i * RES:(N + i) * RES].reshape(N, RES)
         for i in range(2 * (KNN + 1))], axis=1)

    args = [h_paras.reshape(N, 1), e0s,
            nW0, nb0.reshape(1, -1), nW1, nb1.reshape(1, -1),
            nW2, nb2.reshape(1, -1),
            cW0, cb0.reshape(1, -1), cW1, cb1.reshape(1, -1),
            cW2, cb2.reshape(1, -1),
            kW0, kb0.reshape(1, -1), kW1, kb1.reshape(1, -1),
            kW2, kb2.reshape(1, -1),
            eW0, eb0.reshape(1, -1), eW1, eb1.reshape(1, -1),
            eW2, eb2.reshape(1, -1)]

    acc = pl.pallas_call(
        _body,
        out_shape=jax.ShapeDtypeStruct((N + 5, RES), jnp.float32),
    )(*args)

    start = (2 * KNN + 1) * RES // 2
    return acc.reshape(-1)[start:start + N * RES]


# speedup vs baseline: 52.2326x; 1.0405x over previous
import functools

import jax
import jax.numpy as jnp
import numpy as np
from jax.experimental import pallas as pl

N = 1000
KNN = 2
RES = 50
MODES = 2
N0 = 1.5
C_EPSILON = 1.0
DX = 0.01
DH = 0.1
H_MIN = 0.2
H_MAX = 0.8
EY_SIZE = 2 * (KNN + 1) * RES
M = MODES * N
NBLK = M // 8
NBLKP = 256
OFFSETS = (-3, -2, -1, 1, 2, 3)

_F32 = jnp.float32
_HIGH = jax.lax.Precision.HIGHEST


def _dot(a, b):
    return jnp.dot(a, b, preferred_element_type=_F32, precision=_HIGH)


def _dotb(a, b):
    return jnp.dot(a.astype(jnp.bfloat16), b.astype(jnp.bfloat16),
                   preferred_element_type=_F32)


def _clampshift(col, o):
    if o == 0:
        return col
    if o > 0:
        return jnp.concatenate(
            [col[o:], jnp.broadcast_to(col[N - 1:N], (o, 1))], axis=0)
    return jnp.concatenate(
        [jnp.broadcast_to(col[0:1], (-o, 1)), col[:N + o]], axis=0)


def _bmm(a, b):
    out = a[:, :, 0:1] * b[:, 0:1, :]
    for k in range(1, 8):
        out = out + a[:, :, k:k + 1] * b[:, k:k + 1, :]
    return out


def _bmv(a, v):
    return jnp.sum(a * v[:, None, :], axis=2)


def _eye8():
    r = jax.lax.broadcasted_iota(jnp.int32, (8, 8), 0)
    c = jax.lax.broadcasted_iota(jnp.int32, (8, 8), 1)
    return (r == c).astype(_F32)


def _binv(d):
    bsz = d.shape[0]
    eye8 = _eye8()
    aug = jnp.concatenate([d, jnp.broadcast_to(eye8, (bsz, 8, 8))], axis=2)
    for p in range(8):
        ep = eye8[p]
        piv = aug[:, p, p]
        pivrow = aug[:, p, :] * (1.0 / piv)[:, None]
        fac = aug[:, :, p] - ep[None, :]
        aug = aug - fac[:, :, None] * pivrow[:, None, :]
    return aug[:, :, 8:]


def _shift_down1(x):
    if x.shape[0] == 1:
        return jnp.zeros_like(x)
    return jnp.concatenate([jnp.zeros_like(x[0:1]), x[:-1]], axis=0)


def _shift_up1(x):
    if x.shape[0] == 1:
        return jnp.zeros_like(x)
    return jnp.concatenate([x[1:], jnp.zeros_like(x[0:1])], axis=0)


def _body(h_ref, e0s_ref,
          nW0, nb0, nW1, nb1, nW2, nb2,
          cW0, cb0, cW1, cb1, cW2, cb2,
          kW0, kb0, kW1, kb1, kW2, kb2,
          eW0, eb0, eW1, eb1, eW2, eb2,
          out_ref):
    hs = jax.nn.sigmoid(h_ref[:]) * (H_MAX - H_MIN) + H_MIN

    h = jnp.tanh(hs * nW0[:] + nb0[:])
    h = jnp.tanh(_dotb(h, nW1[:]) + nb1[:])
    neff = _dotb(h, nW2[:]) + nb2[:]

    h = jnp.tanh(hs * eW0[:] + eb0[:])
    h = jnp.tanh(_dotb(h, eW1[:]) + eb1[:])
    ey = _dotb(h, eW2[:]) + eb2[:]
    ey0 = ey[:, :EY_SIZE]
    ey1 = ey[:, EY_SIZE:]
    e0s = e0s_ref[:]
    es0 = jnp.sum(ey0 * e0s, axis=1, keepdims=True)
    es1 = jnp.sum(ey1 * e0s, axis=1, keepdims=True)
    e_sum = jnp.concatenate([es0, es1], axis=1)
    eta = neff * N0 / (neff + N0)
    u0 = (2.0 * C_EPSILON * DX) * eta * e_sum

    hsh = [_clampshift(hs, o) for o in OFFSETS]

    def edge_mlp(W0, b0, W1, b1, W2, b2):
        xs = []
        for oi, o in enumerate(OFFSETS):
            pre = hs * W0[0:1, :] + hsh[oi] * W0[1:2, :] \
                + (float(o) * W0[2:3, :] + b0[:])
            xs.append(jnp.tanh(pre))
        x = jnp.concatenate(xs, axis=0)
        x = jnp.tanh(_dotb(x, W1[:]) + b1[:])
        v = _dotb(x, W2[:]) + b2[:]
        return [v[oi * N:(oi + 1) * N] for oi in range(len(OFFSETS))]

    cv = edge_mlp(cW0, cb0, cW1, cb1, cW2, cb2)
    kv = edge_mlp(kW0, kb0, kW1, kb1, kW2, kb2)

    ycols = [jnp.zeros((N, 1), _F32), jnp.zeros((N, 1), _F32)]
    ucols = [u0[:, 0:1], u0[:, 1:2]]
    for oi, o in enumerate(OFFSETS):
        for mi in range(2):
            for mj in range(2):
                ycols[mi] = ycols[mi] + kv[oi][:, 2 * mi + mj:2 * mi + mj + 1] \
                    * _clampshift(ucols[mj], o)
    yk = jnp.concatenate(ycols, axis=1)

    def _pick4(x, s):
        w = x.shape[1]
        return x.reshape(NBLK, 4, w)[:, s, :]

    b = jnp.concatenate([_pick4(yk, s) for s in range(4)], axis=1)

    zcol = jnp.zeros((N, 1), _F32)
    cols = {}
    for oi, o in enumerate(OFFSETS):
        for mi in range(2):
            for mj in range(2):
                key = (mi, 2 * o + mj - mi + 7)
                v = cv[oi][:, 2 * mi + mj:2 * mi + mj + 1]
                cols[key] = cols[key] + v if key in cols else v
    band_mi = []
    for mi in range(2):
        band_mi.append(jnp.concatenate(
            [cols.get((mi, di), zcol) for di in range(15)], axis=1))

    drows, lrows, urows = [], [], []
    for a in range(8):
        ba = _pick4(band_mi[a % 2], a // 2)
        drows.append(ba[:, 7 - a:15 - a])
        lw = 7 - a
        if lw <= 0:
            lrows.append(jnp.zeros((NBLK, 8), _F32))
        else:
            lrows.append(jnp.concatenate(
                [jnp.zeros((NBLK, a + 1), _F32), ba[:, 0:lw]], axis=1))
        if a == 0:
            urows.append(jnp.zeros((NBLK, 8), _F32))
        else:
            urows.append(jnp.concatenate(
                [ba[:, 15 - a:15], jnp.zeros((NBLK, 8 - a), _F32)], axis=1))
    eye8 = _eye8()
    dm = jnp.stack(drows, axis=1) + eye8[None]
    lm = jnp.stack(lrows, axis=1)
    um = jnp.stack(urows, axis=1)

    def _r22(x):
        return jnp.concatenate([x[:, 0:2], x[:, 2:4]], axis=0)

    pb0 = _r22(cv[0][0:1] + cv[1][0:1] + cv[2][0:1])
    pb1 = _r22(cv[0][1:2] + cv[1][1:2])
    pb2 = _r22(cv[0][2:3])
    pbot = jnp.concatenate([pb0, pb1, pb2], axis=0)
    patch0 = jnp.concatenate(
        [jnp.concatenate([pbot, jnp.zeros((6, 6), _F32)], axis=1),
         jnp.zeros((2, 8), _F32)], axis=0)
    pt0 = _r22(cv[5][N - 3:N - 2])
    pt1 = _r22(cv[4][N - 2:N - 1] + cv[5][N - 2:N - 1])
    pt2 = _r22(cv[3][N - 1:N] + cv[4][N - 1:N] + cv[5][N - 1:N])
    ptop = jnp.concatenate([pt0, pt1, pt2], axis=0)
    patch9 = jnp.concatenate(
        [jnp.zeros((2, 8), _F32),
         jnp.concatenate([jnp.zeros((6, 6), _F32), ptop], axis=1)], axis=0)
    dm = jnp.concatenate(
        [dm[0:1] + patch0[None], dm[1:NBLK - 1], dm[NBLK - 1:] + patch9[None]],
        axis=0)

    pad = NBLKP - NBLK
    dm = jnp.concatenate([dm, jnp.broadcast_to(eye8, (pad, 8, 8))], axis=0)
    lm = jnp.concatenate([lm, jnp.zeros((pad, 8, 8), _F32)], axis=0)
    um = jnp.concatenate([um, jnp.zeros((pad, 8, 8), _F32)], axis=0)
    b = jnp.concatenate([b, jnp.zeros((pad, 8), _F32)], axis=0)
    lm = jnp.concatenate([jnp.zeros((1, 8, 8), _F32), lm[1:]], axis=0)
    um = jnp.concatenate(
        [um[:NBLK - 1], jnp.zeros((1, 8, 8), _F32), um[NBLK:]], axis=0)

    levels = []
    while dm.shape[0] > 1:
        half = dm.shape[0] // 2
        dr = dm.reshape(half, 2, 8, 8)
        lr = lm.reshape(half, 2, 8, 8)
        ur = um.reshape(half, 2, 8, 8)
        br = b.reshape(half, 2, 8)
        ed, od = dr[:, 0], dr[:, 1]
        el, ol = lr[:, 0], lr[:, 1]
        eu, ou = ur[:, 0], ur[:, 1]
        eb, ob = br[:, 0], br[:, 1]
        inv_o = _binv(od)
        p = _bmm(el, _shift_down1(inv_o))
        q = _bmm(eu, inv_o)
        dm = ed - _bmm(p, _shift_down1(ou)) - _bmm(q, ol)
        lm = -_bmm(p, _shift_down1(ol))
        um = -_bmm(q, ou)
        b = eb - _bmv(p, _shift_down1(ob)) - _bmv(q, ob)
        levels.append((ol, ou, ob, inv_o))
    x = _bmv(_binv(dm), b)
    for (ol, ou, ob, inv_o) in reversed(levels):
        rhs = ob - _bmv(ol, x) - _bmv(ou, _shift_up1(x))
        xo = _bmv(inv_o, rhs)
        x = jnp.stack([x, xo], axis=1).reshape(-1, 8)
    xr = x[:NBLK]
    rep8 = jnp.repeat(xr, 4, axis=0)
    rmod = jax.lax.broadcasted_iota(jnp.int32, (N, 8), 0) % 4
    lane = jax.lax.broadcasted_iota(jnp.int32, (N, 8), 1)
    du_cols = []
    for mi in range(2):
        mask = (lane == 2 * rmod + mi).astype(_F32)
        du_cols.append(jnp.sum(rep8 * mask, axis=1, keepdims=True))
    du = jnp.concatenate(du_cols, axis=1)

    u = u0 + DH * du

    w = eta * u
    contrib = w[:, 0:1] * ey0 + w[:, 1:2] * ey1
    acc = jnp.zeros((N + 5, RES), _F32)
    for t in range(6):
        piece = contrib[:, t * RES:(t + 1) * RES]
        parts = []
        if t > 0:
            parts.append(jnp.zeros((t, RES), _F32))
        parts.append(piece)
        if t < 5:
            parts.append(jnp.zeros((5 - t, RES), _F32))
        acc = acc + jnp.concatenate(parts, axis=0)
    out_ref[:] = acc


@jax.jit
def kernel(h_paras, E0, nW0, nb0, nW1, nb1, nW2, nb2,
           cW0, cb0, cW1, cb1, cW2, cb2,
           kW0, kb0, kW1, kb1, kW2, kb2,
           eW0, eb0, eW1, eb1, eW2, eb2):
    pad1 = (2 * KNN + 1) * RES // 2
    e0p = jnp.pad(E0, (pad1, (2 * KNN + 1) * RES - pad1))
    e0s = jnp.concatenate(
        [e0p[i * RES:(N + i) * RES].reshape(N, RES)
         for i in range(2 * (KNN + 1))], axis=1)

    args = [h_paras.reshape(N, 1), e0s,
            nW0, nb0.reshape(1, -1), nW1, nb1.reshape(1, -1),
            nW2, nb2.reshape(1, -1),
            cW0, cb0.reshape(1, -1), cW1, cb1.reshape(1, -1),
            cW2, cb2.reshape(1, -1),
            kW0, kb0.reshape(1, -1), kW1, kb1.reshape(1, -1),
            kW2, kb2.reshape(1, -1),
            eW0, eb0.reshape(1, -1), eW1, eb1.reshape(1, -1),
            eW2, eb2.reshape(1, -1)]

    acc = pl.pallas_call(
        _body,
        out_shape=jax.ShapeDtypeStruct((N + 5, RES), jnp.float32),
    )(*args)

    start = (2 * KNN + 1) * RES // 2
    return acc.reshape(-1)[start:start + N * RES]


# speedup vs baseline: 59.2741x; 1.1348x over previous
import functools

import jax
import jax.numpy as jnp
import numpy as np
from jax.experimental import pallas as pl

N = 1000
KNN = 2
RES = 50
MODES = 2
N0 = 1.5
C_EPSILON = 1.0
DX = 0.01
DH = 0.1
H_MIN = 0.2
H_MAX = 0.8
EY_SIZE = 2 * (KNN + 1) * RES
M = MODES * N
NBLK = M // 8
NBLKP = 256
OFFSETS = (-3, -2, -1, 1, 2, 3)

_F32 = jnp.float32
_HIGH = jax.lax.Precision.HIGHEST


def _dot(a, b):
    return jnp.dot(a, b, preferred_element_type=_F32, precision=_HIGH)


def _dotb(a, b):
    return jnp.dot(a.astype(jnp.bfloat16), b.astype(jnp.bfloat16),
                   preferred_element_type=_F32)


def _clampshift(col, o):
    if o == 0:
        return col
    if o > 0:
        return jnp.concatenate(
            [col[o:], jnp.broadcast_to(col[N - 1:N], (o, 1))], axis=0)
    return jnp.concatenate(
        [jnp.broadcast_to(col[0:1], (-o, 1)), col[:N + o]], axis=0)


def _bmm(a, b):
    out = a[:, :, 0:1] * b[:, 0:1, :]
    for k in range(1, 8):
        out = out + a[:, :, k:k + 1] * b[:, k:k + 1, :]
    return out


def _bmv(a, v):
    return jnp.sum(a * v[:, None, :], axis=2)


def _eye8():
    r = jax.lax.broadcasted_iota(jnp.int32, (8, 8), 0)
    c = jax.lax.broadcasted_iota(jnp.int32, (8, 8), 1)
    return (r == c).astype(_F32)


def _bsolve(d, rhs):
    eye8 = _eye8()
    aug = jnp.concatenate([d, rhs], axis=2)
    for p in range(8):
        ep = eye8[p]
        piv = aug[:, p, p]
        pivrow = aug[:, p, :] * (1.0 / piv)[:, None]
        fac = aug[:, :, p] - ep[None, :]
        aug = aug - fac[:, :, None] * pivrow[:, None, :]
    return aug[:, :, 8:]


def _shift_down1(x):
    if x.shape[0] == 1:
        return jnp.zeros_like(x)
    return jnp.concatenate([jnp.zeros_like(x[0:1]), x[:-1]], axis=0)


def _shift_up1(x):
    if x.shape[0] == 1:
        return jnp.zeros_like(x)
    return jnp.concatenate([x[1:], jnp.zeros_like(x[0:1])], axis=0)


def _body(h_ref, e0s_ref,
          nW0, nb0, nW1, nb1, nW2, nb2,
          cW0, cb0, cW1, cb1, cW2, cb2,
          kW0, kb0, kW1, kb1, kW2, kb2,
          eW0, eb0, eW1, eb1, eW2, eb2,
          out_ref):
    hs = jax.nn.sigmoid(h_ref[:]) * (H_MAX - H_MIN) + H_MIN

    h = jnp.tanh(hs * nW0[:] + nb0[:])
    h = jnp.tanh(_dotb(h, nW1[:]) + nb1[:])
    neff = _dotb(h, nW2[:]) + nb2[:]

    h = jnp.tanh(hs * eW0[:] + eb0[:])
    h = jnp.tanh(_dotb(h, eW1[:]) + eb1[:])
    ey = _dotb(h, eW2[:]) + eb2[:]
    ey0 = ey[:, :EY_SIZE]
    ey1 = ey[:, EY_SIZE:]
    e0s = e0s_ref[:]
    es0 = jnp.sum(ey0 * e0s, axis=1, keepdims=True)
    es1 = jnp.sum(ey1 * e0s, axis=1, keepdims=True)
    e_sum = jnp.concatenate([es0, es1], axis=1)
    eta = neff * N0 / (neff + N0)
    u0 = (2.0 * C_EPSILON * DX) * eta * e_sum

    hsh = [_clampshift(hs, o) for o in OFFSETS]

    def edge_mlp(W0, b0, W1, b1, W2, b2):
        xs = []
        for oi, o in enumerate(OFFSETS):
            pre = hs * W0[0:1, :] + hsh[oi] * W0[1:2, :] \
                + (float(o) * W0[2:3, :] + b0[:])
            xs.append(jnp.tanh(pre))
        x = jnp.concatenate(xs, axis=0)
        x = jnp.tanh(_dotb(x, W1[:]) + b1[:])
        v = _dotb(x, W2[:]) + b2[:]
        return [v[oi * N:(oi + 1) * N] for oi in range(len(OFFSETS))]

    cv = edge_mlp(cW0, cb0, cW1, cb1, cW2, cb2)
    kv = edge_mlp(kW0, kb0, kW1, kb1, kW2, kb2)

    ycols = [jnp.zeros((N, 1), _F32), jnp.zeros((N, 1), _F32)]
    ucols = [u0[:, 0:1], u0[:, 1:2]]
    for oi, o in enumerate(OFFSETS):
        for mi in range(2):
            for mj in range(2):
                ycols[mi] = ycols[mi] + kv[oi][:, 2 * mi + mj:2 * mi + mj + 1] \
                    * _clampshift(ucols[mj], o)
    yk = jnp.concatenate(ycols, axis=1)

    def _pick4(x, s):
        w = x.shape[1]
        return x.reshape(NBLK, 4, w)[:, s, :]

    b = jnp.concatenate([_pick4(yk, s) for s in range(4)], axis=1)

    zcol = jnp.zeros((N, 1), _F32)
    cols = {}
    for oi, o in enumerate(OFFSETS):
        for mi in range(2):
            for mj in range(2):
                key = (mi, 2 * o + mj - mi + 7)
                v = cv[oi][:, 2 * mi + mj:2 * mi + mj + 1]
                cols[key] = cols[key] + v if key in cols else v
    band_mi = []
    for mi in range(2):
        band_mi.append(jnp.concatenate(
            [cols.get((mi, di), zcol) for di in range(15)], axis=1))

    drows, lrows, urows = [], [], []
    for a in range(8):
        ba = _pick4(band_mi[a % 2], a // 2)
        drows.append(ba[:, 7 - a:15 - a])
        lw = 7 - a
        if lw <= 0:
            lrows.append(jnp.zeros((NBLK, 8), _F32))
        else:
            lrows.append(jnp.concatenate(
                [jnp.zeros((NBLK, a + 1), _F32), ba[:, 0:lw]], axis=1))
        if a == 0:
            urows.append(jnp.zeros((NBLK, 8), _F32))
        else:
            urows.append(jnp.concatenate(
                [ba[:, 15 - a:15], jnp.zeros((NBLK, 8 - a), _F32)], axis=1))
    eye8 = _eye8()
    dm = jnp.stack(drows, axis=1) + eye8[None]
    lm = jnp.stack(lrows, axis=1)
    um = jnp.stack(urows, axis=1)

    def _r22(x):
        return jnp.concatenate([x[:, 0:2], x[:, 2:4]], axis=0)

    pb0 = _r22(cv[0][0:1] + cv[1][0:1] + cv[2][0:1])
    pb1 = _r22(cv[0][1:2] + cv[1][1:2])
    pb2 = _r22(cv[0][2:3])
    pbot = jnp.concatenate([pb0, pb1, pb2], axis=0)
    patch0 = jnp.concatenate(
        [jnp.concatenate([pbot, jnp.zeros((6, 6), _F32)], axis=1),
         jnp.zeros((2, 8), _F32)], axis=0)
    pt0 = _r22(cv[5][N - 3:N - 2])
    pt1 = _r22(cv[4][N - 2:N - 1] + cv[5][N - 2:N - 1])
    pt2 = _r22(cv[3][N - 1:N] + cv[4][N - 1:N] + cv[5][N - 1:N])
    ptop = jnp.concatenate([pt0, pt1, pt2], axis=0)
    patch9 = jnp.concatenate(
        [jnp.zeros((2, 8), _F32),
         jnp.concatenate([jnp.zeros((6, 6), _F32), ptop], axis=1)], axis=0)
    dm = jnp.concatenate(
        [dm[0:1] + patch0[None], dm[1:NBLK - 1], dm[NBLK - 1:] + patch9[None]],
        axis=0)

    pad = NBLKP - NBLK
    dm = jnp.concatenate([dm, jnp.broadcast_to(eye8, (pad, 8, 8))], axis=0)
    lm = jnp.concatenate([lm, jnp.zeros((pad, 8, 8), _F32)], axis=0)
    um = jnp.concatenate([um, jnp.zeros((pad, 8, 8), _F32)], axis=0)
    b = jnp.concatenate([b, jnp.zeros((pad, 8), _F32)], axis=0)
    lm = jnp.concatenate([jnp.zeros((1, 8, 8), _F32), lm[1:]], axis=0)
    um = jnp.concatenate(
        [um[:NBLK - 1], jnp.zeros((1, 8, 8), _F32), um[NBLK:]], axis=0)

    levels = []
    while dm.shape[0] > 1:
        half = dm.shape[0] // 2
        dr = dm.reshape(half, 2, 8, 8)
        lr = lm.reshape(half, 2, 8, 8)
        ur = um.reshape(half, 2, 8, 8)
        br = b.reshape(half, 2, 8)
        ed, od = dr[:, 0], dr[:, 1]
        el, ol = lr[:, 0], lr[:, 1]
        eu, ou = ur[:, 0], ur[:, 1]
        eb, ob = br[:, 0], br[:, 1]
        sol = _bsolve(od, jnp.concatenate(
            [ou, ol, ob[:, :, None]], axis=2))
        w_e = _bmm(eu, sol)
        w_l = _bmm(el, _shift_down1(sol))
        dm = ed - w_l[:, :, 0:8] - w_e[:, :, 8:16]
        lm = -w_l[:, :, 8:16]
        um = -w_e[:, :, 0:8]
        b = eb - w_l[:, :, 16] - w_e[:, :, 16]
        levels.append((sol[:, :, 0:8], sol[:, :, 8:16], sol[:, :, 16]))
    x = _bsolve(dm, b[:, :, None])[:, :, 0]
    for (a_m, b_m, vb) in reversed(levels):
        xo = vb - _bmv(b_m, x) - _bmv(a_m, _shift_up1(x))
        x = jnp.stack([x, xo], axis=1).reshape(-1, 8)
    xr = x[:NBLK]
    rep8 = jnp.repeat(xr, 4, axis=0)
    rmod = jax.lax.broadcasted_iota(jnp.int32, (N, 8), 0) % 4
    lane = jax.lax.broadcasted_iota(jnp.int32, (N, 8), 1)
    du_cols = []
    for mi in range(2):
        mask = (lane == 2 * rmod + mi).astype(_F32)
        du_cols.append(jnp.sum(rep8 * mask, axis=1, keepdims=True))
    du = jnp.concatenate(du_cols, axis=1)

    u = u0 + DH * du

    w = eta * u
    contrib = w[:, 0:1] * ey0 + w[:, 1:2] * ey1
    acc = jnp.zeros((N + 5, RES), _F32)
    for t in range(6):
        piece = contrib[:, t * RES:(t + 1) * RES]
        parts = []
        if t > 0:
            parts.append(jnp.zeros((t, RES), _F32))
        parts.append(piece)
        if t < 5:
            parts.append(jnp.zeros((5 - t, RES), _F32))
        acc = acc + jnp.concatenate(parts, axis=0)
    out_ref[:] = acc


@jax.jit
def kernel(h_paras, E0, nW0, nb0, nW1, nb1, nW2, nb2,
           cW0, cb0, cW1, cb1, cW2, cb2,
           kW0, kb0, kW1, kb1, kW2, kb2,
           eW0, eb0, eW1, eb1, eW2, eb2):
    pad1 = (2 * KNN + 1) * RES // 2
    e0p = jnp.pad(E0, (pad1, (2 * KNN + 1) * RES - pad1))
    e0s = jnp.concatenate(
        [e0p[i * RES:(N + i) * RES].reshape(N, RES)
         for i in range(2 * (KNN + 1))], axis=1)

    args = [h_paras.reshape(N, 1), e0s,
            nW0, nb0.reshape(1, -1), nW1, nb1.reshape(1, -1),
            nW2, nb2.reshape(1, -1),
            cW0, cb0.reshape(1, -1), cW1, cb1.reshape(1, -1),
            cW2, cb2.reshape(1, -1),
            kW0, kb0.reshape(1, -1), kW1, kb1.reshape(1, -1),
            kW2, kb2.reshape(1, -1),
            eW0, eb0.reshape(1, -1), eW1, eb1.reshape(1, -1),
            eW2, eb2.reshape(1, -1)]

    acc = pl.pallas_call(
        _body,
        out_shape=jax.ShapeDtypeStruct((N + 5, RES), jnp.float32),
    )(*args)

    start = (2 * KNN + 1) * RES // 2
    return acc.reshape(-1)[start:start + N * RES]
